# Initial kernel scaffold; baseline (speedup 1.0000x reference)
import jax
import jax.numpy as jnp
from jax.experimental import pallas as pl


def kernel(x, edge_index, W1, b1, W2, b2):
    raise NotImplementedError("write your pallas kernel here")


# speedup vs baseline: 23.5991x; 23.5991x over previous
import functools

import jax
import jax.numpy as jnp
from jax import lax
from jax.experimental import pallas as pl
from jax.experimental.pallas import tpu as pltpu
from jax.experimental.pallas import tpu_sc as plsc

N = 100000
E = 1600000
NP = 100352
EPAD = 1638400
ER = EPAD // 128
K = 8
WT = NP // 16
D = 16
R = 1024
F32 = jnp.float32

_mesh = plsc.VectorSubcoreMesh(core_axis_name="c", subcore_axis_name="s")


@functools.partial(
    pl.kernel,
    out_type=jax.ShapeDtypeStruct((2, NP), F32),
    mesh=_mesh,
    scratch_types=[
        pltpu.VMEM((K, 128), jnp.int32),
        pltpu.VMEM((2048,), F32),
        pltpu.VMEM((128,), F32),
        pltpu.VMEM_SHARED((NP,), F32),
    ],
)
def _deg_kernel(dst_hbm, out_hbm, dstv, zbuf, ones_v, acc):
    cid = lax.axis_index("c")
    sid = lax.axis_index("s")

    def _z16(i, c):
        zbuf[pl.ds(i * 16, 16)] = jnp.zeros((16,), F32)
        return c

    lax.fori_loop(0, 128, _z16, 0)

    def _o16(i, c):
        ones_v[pl.ds(i * 16, 16)] = jnp.ones((16,), F32)
        return c

    lax.fori_loop(0, 8, _o16, 0)

    base = sid * WT
    for t in range(3):
        pltpu.sync_copy(zbuf, acc.at[pl.ds(base + t * 2048, 2048)])
    pltpu.sync_copy(zbuf.at[pl.ds(0, 128)], acc.at[pl.ds(base + 3 * 2048, 128)])
    plsc.subcore_barrier()

    rows_per_tile = ER // 32
    row_base = cid * (ER // 2) + sid * rows_per_tile

    def _chunk(ci, c):
        r0 = row_base + ci * K
        pltpu.sync_copy(dst_hbm.at[pl.ds(r0, K)], dstv)
        for j in range(K):
            pltpu.sync_copy(ones_v, acc.at[dstv.at[j]], add=True)
        return c

    lax.fori_loop(0, rows_per_tile // K, _chunk, 0)

    plsc.subcore_barrier()
    pltpu.sync_copy(acc.at[pl.ds(base, WT)], out_hbm.at[cid, pl.ds(base, WT)])


def _make_prop(split_edges):
    @functools.partial(
        pl.kernel,
        out_type=jax.ShapeDtypeStruct((2, NP, D), F32),
        mesh=_mesh,
        scratch_types=[
            pltpu.VMEM((K, 128), jnp.int32),
            pltpu.VMEM((K, 128), jnp.int32),
            pltpu.VMEM((K * 128, D), F32),
            pltpu.VMEM_SHARED((NP, D), F32),
            pltpu.SemaphoreType.DMA,
        ],
        compiler_params=pltpu.CompilerParams(use_tc_tiling_on_sc=False),
    )
    def prop(src_hbm, dst_hbm, table_hbm, out_hbm, srcv, dstv, rows, acc, sem):
        cid = lax.axis_index("c")
        sid = lax.axis_index("s")

        def _z(i, c):
            rows[i, :] = jnp.zeros((D,), F32)
            return c

        lax.fori_loop(0, K * 128, _z, 0)

        base = sid * WT
        zr = K * 128
        for t in range(WT // zr):
            pltpu.sync_copy(rows, acc.at[pl.ds(base + t * zr, zr)])
        if WT % zr:
            pltpu.sync_copy(rows.at[pl.ds(0, WT % zr)],
                            acc.at[pl.ds(base + (WT // zr) * zr, WT % zr)])
        plsc.subcore_barrier()

        if split_edges:
            rows_per_tile = ER // 32
            row_base = cid * (ER // 2) + sid * rows_per_tile

            def src_slice(r0):
                return src_hbm.at[pl.ds(r0, K)]
        else:
            rows_per_tile = ER // 16
            row_base = sid * rows_per_tile

            def src_slice(r0):
                return src_hbm.at[cid, pl.ds(r0, K)]

        def _chunk(ci, c):
            r0 = row_base + ci * K
            pltpu.sync_copy(src_slice(r0), srcv)
            pltpu.sync_copy(dst_hbm.at[pl.ds(r0, K)], dstv)
            cps = [
                pltpu.async_copy(table_hbm.at[srcv.at[j]],
                                 rows.at[pl.ds(j * 128, 128)], sem)
                for j in range(K)
            ]
            for cp in cps:
                cp.wait()
            for j in range(K):
                pltpu.sync_copy(rows.at[pl.ds(j * 128, 128)],
                                acc.at[dstv.at[j]], add=True)
            return c

        lax.fori_loop(0, rows_per_tile // K, _chunk, 0)

        plsc.subcore_barrier()
        pltpu.sync_copy(acc.at[pl.ds(base, WT)],
                        out_hbm.at[cid, pl.ds(base, WT)])

    return prop


_prop1 = _make_prop(split_edges=False)
_prop2 = _make_prop(split_edges=True)


def _tc_scale(xpad, degp):
    def body(x_ref, d_ref, xs2_ref, dinv_ref):
        deg = d_ref[0, :] + d_ref[1, :] + 1.0
        dinv = lax.rsqrt(deg)
        xs = x_ref[...] * dinv[:, None]
        xs2_ref[0, :, :] = xs[:, :16]
        xs2_ref[1, :, :] = xs[:, 16:]
        dinv_ref[...] = dinv[:, None]

    return pl.pallas_call(
        body,
        grid=(NP // R,),
        in_specs=[
            pl.BlockSpec((R, 32), lambda i: (i, 0)),
            pl.BlockSpec((2, R), lambda i: (0, i)),
        ],
        out_specs=[
            pl.BlockSpec((2, R, 16), lambda i: (0, i, 0)),
            pl.BlockSpec((R, 1), lambda i: (i, 0)),
        ],
        out_shape=[
            jax.ShapeDtypeStruct((2, NP, 16), F32),
            jax.ShapeDtypeStruct((NP, 1), F32),
        ],
    )(xpad, degp)


def _tc_mid(p2, xs2, dinv, w1p, b1r, w2p):
    def body(p_ref, x_ref, d_ref, w1_ref, b1_ref, w2_ref, o_ref):
        t0 = p_ref[0] + x_ref[0]
        t1 = p_ref[1] + x_ref[1]
        tmp = jnp.concatenate([t0, t1], axis=1) * d_ref[...]
        h = jnp.dot(tmp, w1_ref[...], preferred_element_type=F32)
        h = jnp.maximum(h + b1_ref[0, :], 0.0)
        z = jnp.dot(h, w2_ref[...], preferred_element_type=F32)
        o_ref[...] = z * d_ref[...]

    return pl.pallas_call(
        body,
        grid=(NP // R,),
        in_specs=[
            pl.BlockSpec((2, R, 16), lambda i: (0, i, 0)),
            pl.BlockSpec((2, R, 16), lambda i: (0, i, 0)),
            pl.BlockSpec((R, 1), lambda i: (i, 0)),
            pl.BlockSpec((32, 64), lambda i: (0, 0)),
            pl.BlockSpec((1, 64), lambda i: (0, 0)),
            pl.BlockSpec((64, 16), lambda i: (0, 0)),
        ],
        out_specs=pl.BlockSpec((R, 16), lambda i: (i, 0)),
        out_shape=jax.ShapeDtypeStruct((NP, 16), F32),
    )(p2, xs2, dinv, w1p, b1r, w2p)


def _tc_out(q2, zs16, dinv, b2r):
    def body(q_ref, z_ref, d_ref, b_ref, o_ref):
        s = q_ref[0] + q_ref[1] + z_ref[...]
        o_ref[...] = s * d_ref[...] + b_ref[0, :]

    return pl.pallas_call(
        body,
        grid=(NP // R,),
        in_specs=[
            pl.BlockSpec((2, R, 16), lambda i: (0, i, 0)),
            pl.BlockSpec((R, 16), lambda i: (i, 0)),
            pl.BlockSpec((R, 1), lambda i: (i, 0)),
            pl.BlockSpec((1, 16), lambda i: (0, 0)),
        ],
        out_specs=pl.BlockSpec((R, 16), lambda i: (i, 0)),
        out_shape=jax.ShapeDtypeStruct((NP, 16), F32),
    )(q2, zs16, dinv, b2r)


@jax.jit
def _run(x, edge_index, W1, b1, W2, b2):
    src = edge_index[0].astype(jnp.int32)
    dst = edge_index[1].astype(jnp.int32)
    padi = jnp.full((EPAD - E,), N, jnp.int32)
    srcp = jnp.concatenate([src, padi]).reshape(ER, 128)
    dstp = jnp.concatenate([dst, padi]).reshape(ER, 128)
    src2 = jnp.stack([srcp, srcp + NP])

    xpad = jnp.zeros((NP, 32), F32).at[:N, :29].set(x)
    w1p = jnp.zeros((32, 64), F32).at[:29, :].set(W1)
    w2p = jnp.zeros((64, 16), F32).at[:, :2].set(W2)
    b1r = b1.reshape(1, 64)
    b2r = jnp.zeros((1, 16), F32).at[0, :2].set(b2)

    degp = _deg_kernel(dstp)
    xs2, dinv = _tc_scale(xpad, degp)
    xs_tab = xs2.reshape(2 * NP, 16)
    p2 = _prop1(src2, dstp, xs_tab)
    zs16 = _tc_mid(p2, xs2, dinv, w1p, b1r, w2p)
    q2 = _prop2(srcp, dstp, zs16)
    outp = _tc_out(q2, zs16, dinv, b2r)
    return outp[:N, :2]


def kernel(x, edge_index, W1, b1, W2, b2):
    return _run(x, edge_index, W1, b1, W2, b2)


# speedup vs baseline: 26.7185x; 1.1322x over previous
import functools

import jax
import jax.numpy as jnp
from jax import lax
from jax.experimental import pallas as pl
from jax.experimental.pallas import tpu as pltpu
from jax.experimental.pallas import tpu_sc as plsc

N = 100000
E = 1600000
NP = 100352
EPAD = 1638400
ER = EPAD // 128
K = 5
WT = NP // 16
D = 16
R = 1024
F32 = jnp.float32

_mesh = plsc.VectorSubcoreMesh(core_axis_name="c", subcore_axis_name="s")
_sc_params = pltpu.CompilerParams(use_tc_tiling_on_sc=False)


@functools.partial(
    pl.kernel,
    out_type=jax.ShapeDtypeStruct((2, NP), F32),
    mesh=_mesh,
    scratch_types=[
        pltpu.VMEM((3, K, 128), jnp.int32),
        pltpu.VMEM((128,), F32),
        pltpu.VMEM_SHARED((NP,), F32),
        pltpu.SemaphoreType.DMA,
        pltpu.SemaphoreType.DMA,
    ],
    compiler_params=_sc_params,
)
def _deg_kernel(dst_hbm, zeros_hbm, out_hbm, edv, ones_v, acc, isem, ssem):
    cid = lax.axis_index("c")
    sid = lax.axis_index("s")

    def _o16(i, c):
        ones_v[pl.ds(i * 16, 16)] = jnp.ones((16,), F32)
        return c

    lax.fori_loop(0, 8, _o16, 0)

    base = sid * WT
    pltpu.sync_copy(zeros_hbm, acc.at[pl.ds(base, WT)])
    plsc.subcore_barrier()

    rows_per_tile = ER // 32
    nch = rows_per_tile // K
    row_base = cid * (ER // 2) + sid * rows_per_tile

    def start_stage(c, buf):
        pltpu.async_copy(dst_hbm.at[pl.ds(row_base + c * K, K)],
                         edv.at[buf], isem)

    def drain_stage():
        pltpu.make_async_copy(dst_hbm.at[pl.ds(row_base, K)],
                              edv.at[0], isem).wait()

    def start_scatters(buf):
        for j in range(K):
            pltpu.async_copy(ones_v, acc.at[edv.at[buf, j]], ssem, add=True)

    def drain_scatters():
        for j in range(K):
            pltpu.make_async_copy(ones_v, acc.at[edv.at[0, j]], ssem).wait()

    start_stage(0, 0)

    def body(ci, carry):
        cb = lax.rem(ci, 3)
        drain_stage()

        @pl.when(ci >= 2)
        def _():
            drain_scatters()

        @pl.when(ci + 1 < nch)
        def _():
            start_stage(ci + 1, lax.rem(ci + 1, 3))

        start_scatters(cb)
        return carry

    lax.fori_loop(0, nch, body, 0)
    drain_scatters()
    drain_scatters()

    plsc.subcore_barrier()
    pltpu.sync_copy(acc.at[pl.ds(base, WT)], out_hbm.at[cid, pl.ds(base, WT)])


def _make_prop(split_edges):
    @functools.partial(
        pl.kernel,
        out_type=jax.ShapeDtypeStruct((2, NP, D), F32),
        mesh=_mesh,
        scratch_types=[
            pltpu.VMEM((3, K, 2, 128), jnp.int32),
            pltpu.VMEM((2, K * 128, D), F32),
            pltpu.VMEM_SHARED((NP, D), F32),
            pltpu.SemaphoreType.DMA,
            pltpu.SemaphoreType.DMA,
            pltpu.SemaphoreType.DMA,
        ],
        compiler_params=_sc_params,
    )
    def prop(ed_hbm, table_hbm, zeros_hbm, out_hbm,
             edv, rows, acc, isem, gsem, ssem):
        cid = lax.axis_index("c")
        sid = lax.axis_index("s")

        base = sid * WT
        pltpu.sync_copy(zeros_hbm, acc.at[pl.ds(base, WT)])
        plsc.subcore_barrier()

        if split_edges:
            rows_per_tile = ER // 32
            row_base = cid * (ER // 2) + sid * rows_per_tile

            def ed_slice(r0):
                return ed_hbm.at[pl.ds(r0, K)]
        else:
            rows_per_tile = ER // 16
            row_base = sid * rows_per_tile

            def ed_slice(r0):
                return ed_hbm.at[cid, pl.ds(r0, K)]

        nch = rows_per_tile // K

        def start_stage(c, buf):
            pltpu.async_copy(ed_slice(row_base + c * K), edv.at[buf], isem)

        def drain_stage():
            pltpu.make_async_copy(ed_slice(row_base), edv.at[0], isem).wait()

        def start_gathers(buf3, b2):
            for j in range(K):
                pltpu.async_copy(table_hbm.at[edv.at[buf3, j, 0]],
                                 rows.at[b2, pl.ds(j * 128, 128)], gsem)

        def drain_gathers():
            for j in range(K):
                pltpu.make_async_copy(table_hbm.at[edv.at[0, j, 0]],
                                      rows.at[0, pl.ds(j * 128, 128)],
                                      gsem).wait()

        def start_scatters(buf3, b2):
            for j in range(K):
                pltpu.async_copy(rows.at[b2, pl.ds(j * 128, 128)],
                                 acc.at[edv.at[buf3, j, 1]], ssem, add=True)

        def drain_scatters():
            for j in range(K):
                pltpu.make_async_copy(rows.at[0, pl.ds(j * 128, 128)],
                                      acc.at[edv.at[0, j, 1]], ssem).wait()

        start_stage(0, 0)

        def body(ci, carry):
            b = lax.rem(ci, 2)
            cb = lax.rem(ci, 3)
            pb = lax.rem(ci + 1, 2)
            pcb = lax.rem(ci + 2, 3)
            drain_stage()

            @pl.when(ci >= 2)
            def _():
                drain_scatters()

            @pl.when(ci + 1 < nch)
            def _():
                start_stage(ci + 1, lax.rem(ci + 1, 3))

            @pl.when(ci >= 1)
            def _():
                drain_gathers()
                start_scatters(pcb, pb)

            start_gathers(cb, b)
            return carry

        lax.fori_loop(0, nch, body, 0)

        drain_scatters()
        drain_gathers()
        start_scatters((nch - 1) % 3, (nch - 1) % 2)
        drain_scatters()

        plsc.subcore_barrier()
        pltpu.sync_copy(acc.at[pl.ds(base, WT)],
                        out_hbm.at[cid, pl.ds(base, WT)])

    return prop


_prop1 = _make_prop(split_edges=False)
_prop2 = _make_prop(split_edges=True)


def _tc_scale(xpad, degp):
    def body(x_ref, d_ref, xs2_ref, dinv_ref):
        deg = d_ref[0, :] + d_ref[1, :] + 1.0
        dinv = lax.rsqrt(deg)
        xs = x_ref[...] * dinv[:, None]
        xs2_ref[0, :, :] = xs[:, :16]
        xs2_ref[1, :, :] = xs[:, 16:]
        dinv_ref[...] = dinv[:, None]

    return pl.pallas_call(
        body,
        grid=(NP // R,),
        in_specs=[
            pl.BlockSpec((R, 32), lambda i: (i, 0)),
            pl.BlockSpec((2, R), lambda i: (0, i)),
        ],
        out_specs=[
            pl.BlockSpec((2, R, 16), lambda i: (0, i, 0)),
            pl.BlockSpec((R, 1), lambda i: (i, 0)),
        ],
        out_shape=[
            jax.ShapeDtypeStruct((2, NP, 16), F32),
            jax.ShapeDtypeStruct((NP, 1), F32),
        ],
    )(xpad, degp)


def _tc_mid(p2, xs2, dinv, w1p, b1r, w2p):
    def body(p_ref, x_ref, d_ref, w1_ref, b1_ref, w2_ref, o_ref):
        t0 = p_ref[0] + x_ref[0]
        t1 = p_ref[1] + x_ref[1]
        tmp = jnp.concatenate([t0, t1], axis=1) * d_ref[...]
        h = jnp.dot(tmp, w1_ref[...], preferred_element_type=F32)
        h = jnp.maximum(h + b1_ref[0, :], 0.0)
        z = jnp.dot(h, w2_ref[...], preferred_element_type=F32)
        o_ref[...] = z * d_ref[...]

    return pl.pallas_call(
        body,
        grid=(NP // R,),
        in_specs=[
            pl.BlockSpec((2, R, 16), lambda i: (0, i, 0)),
            pl.BlockSpec((2, R, 16), lambda i: (0, i, 0)),
            pl.BlockSpec((R, 1), lambda i: (i, 0)),
            pl.BlockSpec((32, 64), lambda i: (0, 0)),
            pl.BlockSpec((1, 64), lambda i: (0, 0)),
            pl.BlockSpec((64, 16), lambda i: (0, 0)),
        ],
        out_specs=pl.BlockSpec((R, 16), lambda i: (i, 0)),
        out_shape=jax.ShapeDtypeStruct((NP, 16), F32),
    )(p2, xs2, dinv, w1p, b1r, w2p)


def _tc_out(q2, zs16, dinv, b2r):
    def body(q_ref, z_ref, d_ref, b_ref, o_ref):
        s = q_ref[0] + q_ref[1] + z_ref[...]
        o_ref[...] = s * d_ref[...] + b_ref[0, :]

    return pl.pallas_call(
        body,
        grid=(NP // R,),
        in_specs=[
            pl.BlockSpec((2, R, 16), lambda i: (0, i, 0)),
            pl.BlockSpec((R, 16), lambda i: (i, 0)),
            pl.BlockSpec((R, 1), lambda i: (i, 0)),
            pl.BlockSpec((1, 16), lambda i: (0, 0)),
        ],
        out_specs=pl.BlockSpec((R, 16), lambda i: (i, 0)),
        out_shape=jax.ShapeDtypeStruct((NP, 16), F32),
    )(q2, zs16, dinv, b2r)


@jax.jit
def _run(x, edge_index, W1, b1, W2, b2):
    src = edge_index[0].astype(jnp.int32)
    dst = edge_index[1].astype(jnp.int32)
    pad_src = jnp.full((EPAD - E,), N, jnp.int32)
    pad_dst = N + jnp.arange(EPAD - E, dtype=jnp.int32) % (NP - N)
    srcp = jnp.concatenate([src, pad_src]).reshape(ER, 128)
    dstp = jnp.concatenate([dst, pad_dst]).reshape(ER, 128)
    ed2 = jnp.stack([srcp, dstp], axis=1)
    ed1 = jnp.stack([ed2, jnp.stack([srcp + NP, dstp], axis=1)])

    xpad = jnp.zeros((NP, 32), F32).at[:N, :29].set(x)
    w1p = jnp.zeros((32, 64), F32).at[:29, :].set(W1)
    w2p = jnp.zeros((64, 16), F32).at[:, :2].set(W2)
    b1r = b1.reshape(1, 64)
    b2r = jnp.zeros((1, 16), F32).at[0, :2].set(b2)
    zrow = jnp.zeros((WT, D), F32)
    zdeg = jnp.zeros((WT,), F32)

    degp = _deg_kernel(dstp, zdeg)
    xs2, dinv = _tc_scale(xpad, degp)
    xs_tab = xs2.reshape(2 * NP, 16)
    p2 = _prop1(ed1, xs_tab, zrow)
    zs16 = _tc_mid(p2, xs2, dinv, w1p, b1r, w2p)
    q2 = _prop2(ed2, zs16, zrow)
    outp = _tc_out(q2, zs16, dinv, b2r)
    return outp[:N, :2]


def kernel(x, edge_index, W1, b1, W2, b2):
    return _run(x, edge_index, W1, b1, W2, b2)


# speedup vs baseline: 36.6276x; 1.3709x over previous
import functools

import jax
import jax.numpy as jnp
from jax import lax
from jax.experimental import pallas as pl
from jax.experimental.pallas import tpu as pltpu
from jax.experimental.pallas import tpu_sc as plsc

N = 100000
E = 1600000
NP = 100352
EPAD = 1638400
ER = EPAD // 128
K = 5
WT = NP // 16
D = 16
R = 1024
F32 = jnp.float32

_mesh = plsc.VectorSubcoreMesh(core_axis_name="c", subcore_axis_name="s")
_sc_params = pltpu.CompilerParams(use_tc_tiling_on_sc=False)


@functools.partial(
    pl.kernel,
    out_type=jax.ShapeDtypeStruct((2, NP), F32),
    mesh=_mesh,
    scratch_types=[
        pltpu.VMEM((3, K, 128), jnp.int32),
        pltpu.VMEM((128,), F32),
        pltpu.VMEM_SHARED((NP,), F32),
        pltpu.SemaphoreType.DMA,
        pltpu.SemaphoreType.DMA,
    ],
    compiler_params=_sc_params,
)
def _deg_kernel(dst_hbm, zeros_hbm, out_hbm, edv, ones_v, acc, isem, ssem):
    cid = lax.axis_index("c")
    sid = lax.axis_index("s")

    def _o16(i, c):
        ones_v[pl.ds(i * 16, 16)] = jnp.ones((16,), F32)
        return c

    lax.fori_loop(0, 8, _o16, 0)

    base = sid * WT
    pltpu.sync_copy(zeros_hbm, acc.at[pl.ds(base, WT)])
    plsc.subcore_barrier()

    rows_per_tile = ER // 32
    nch = rows_per_tile // K
    row_base = cid * (ER // 2) + sid * rows_per_tile

    def start_stage(c, buf):
        pltpu.async_copy(dst_hbm.at[pl.ds(row_base + c * K, K)],
                         edv.at[buf], isem)

    def drain_stage():
        pltpu.make_async_copy(dst_hbm.at[pl.ds(row_base, K)],
                              edv.at[0], isem).wait()

    def start_scatters(buf):
        for j in range(K):
            pltpu.async_copy(ones_v, acc.at[edv.at[buf, j]], ssem, add=True)

    def drain_scatters():
        pltpu.make_async_copy(dst_hbm.at[pl.ds(row_base, K)],
                              edv.at[0], ssem).wait()

    start_stage(0, 0)

    def body(ci, carry):
        cb = lax.rem(ci, 3)
        drain_stage()

        @pl.when(ci >= 2)
        def _():
            drain_scatters()

        @pl.when(ci + 1 < nch)
        def _():
            start_stage(ci + 1, lax.rem(ci + 1, 3))

        start_scatters(cb)
        return carry

    lax.fori_loop(0, nch, body, 0)
    drain_scatters()
    drain_scatters()

    plsc.subcore_barrier()
    pltpu.sync_copy(acc.at[pl.ds(base, WT)], out_hbm.at[cid, pl.ds(base, WT)])


def _make_prop(split_edges):
    @functools.partial(
        pl.kernel,
        out_type=jax.ShapeDtypeStruct((2, NP, D), F32),
        mesh=_mesh,
        scratch_types=[
            pltpu.VMEM((3, K, 2, 128), jnp.int32),
            pltpu.VMEM((2, K * 128, D), F32),
            pltpu.VMEM_SHARED((NP, D), F32),
            pltpu.SemaphoreType.DMA,
            pltpu.SemaphoreType.DMA,
            pltpu.SemaphoreType.DMA,
        ],
        compiler_params=_sc_params,
    )
    def prop(ed_hbm, table_hbm, zeros_hbm, out_hbm,
             edv, rows, acc, isem, gsem, ssem):
        cid = lax.axis_index("c")
        sid = lax.axis_index("s")

        base = sid * WT
        pltpu.sync_copy(zeros_hbm, acc.at[pl.ds(base, WT)])
        plsc.subcore_barrier()

        if split_edges:
            rows_per_tile = ER // 32
            row_base = cid * (ER // 2) + sid * rows_per_tile

            def ed_slice(r0):
                return ed_hbm.at[pl.ds(r0, K)]
        else:
            rows_per_tile = ER // 16
            row_base = sid * rows_per_tile

            def ed_slice(r0):
                return ed_hbm.at[cid, pl.ds(r0, K)]

        nch = rows_per_tile // K

        def start_stage(c, buf):
            pltpu.async_copy(ed_slice(row_base + c * K), edv.at[buf], isem)

        def drain_stage():
            pltpu.make_async_copy(ed_slice(row_base), edv.at[0], isem).wait()

        def start_gathers(buf3, b2):
            for j in range(K):
                pltpu.async_copy(table_hbm.at[edv.at[buf3, j, 0]],
                                 rows.at[b2, pl.ds(j * 128, 128)], gsem)

        def drain_gathers():
            pltpu.make_async_copy(table_hbm.at[pl.ds(0, K * 128)],
                                  rows.at[0], gsem).wait()

        def start_scatters(buf3, b2):
            for j in range(K):
                pltpu.async_copy(rows.at[b2, pl.ds(j * 128, 128)],
                                 acc.at[edv.at[buf3, j, 1]], ssem, add=True)

        def drain_scatters():
            pltpu.make_async_copy(table_hbm.at[pl.ds(0, K * 128)],
                                  rows.at[0], ssem).wait()

        start_stage(0, 0)

        def body(ci, carry):
            b = lax.rem(ci, 2)
            cb = lax.rem(ci, 3)
            pb = lax.rem(ci + 1, 2)
            pcb = lax.rem(ci + 2, 3)
            drain_stage()

            @pl.when(ci >= 2)
            def _():
                drain_scatters()

            @pl.when(ci + 1 < nch)
            def _():
                start_stage(ci + 1, lax.rem(ci + 1, 3))

            @pl.when(ci >= 1)
            def _():
                drain_gathers()
                start_scatters(pcb, pb)

            start_gathers(cb, b)
            return carry

        lax.fori_loop(0, nch, body, 0)

        drain_scatters()
        drain_gathers()
        start_scatters((nch - 1) % 3, (nch - 1) % 2)
        drain_scatters()

        plsc.subcore_barrier()
        pltpu.sync_copy(acc.at[pl.ds(base, WT)],
                        out_hbm.at[cid, pl.ds(base, WT)])

    return prop


_prop1 = _make_prop(split_edges=False)
_prop2 = _make_prop(split_edges=True)


def _tc_scale(xpad, degp):
    def body(x_ref, d_ref, xs2_ref, dinv_ref):
        deg = d_ref[0, :] + d_ref[1, :] + 1.0
        dinv = lax.rsqrt(deg)
        xs = x_ref[...] * dinv[:, None]
        xs2_ref[0, :, :] = xs[:, :16]
        xs2_ref[1, :, :] = xs[:, 16:]
        dinv_ref[...] = dinv[:, None]

    return pl.pallas_call(
        body,
        grid=(NP // R,),
        in_specs=[
            pl.BlockSpec((R, 32), lambda i: (i, 0)),
            pl.BlockSpec((2, R), lambda i: (0, i)),
        ],
        out_specs=[
            pl.BlockSpec((2, R, 16), lambda i: (0, i, 0)),
            pl.BlockSpec((R, 1), lambda i: (i, 0)),
        ],
        out_shape=[
            jax.ShapeDtypeStruct((2, NP, 16), F32),
            jax.ShapeDtypeStruct((NP, 1), F32),
        ],
    )(xpad, degp)


def _tc_mid(p2, xs2, dinv, w1p, b1r, w2p):
    def body(p_ref, x_ref, d_ref, w1_ref, b1_ref, w2_ref, o_ref):
        t0 = p_ref[0] + x_ref[0]
        t1 = p_ref[1] + x_ref[1]
        tmp = jnp.concatenate([t0, t1], axis=1) * d_ref[...]
        h = jnp.dot(tmp, w1_ref[...], preferred_element_type=F32)
        h = jnp.maximum(h + b1_ref[0, :], 0.0)
        z = jnp.dot(h, w2_ref[...], preferred_element_type=F32)
        o_ref[...] = z * d_ref[...]

    return pl.pallas_call(
        body,
        grid=(NP // R,),
        in_specs=[
            pl.BlockSpec((2, R, 16), lambda i: (0, i, 0)),
            pl.BlockSpec((2, R, 16), lambda i: (0, i, 0)),
            pl.BlockSpec((R, 1), lambda i: (i, 0)),
            pl.BlockSpec((32, 64), lambda i: (0, 0)),
            pl.BlockSpec((1, 64), lambda i: (0, 0)),
            pl.BlockSpec((64, 16), lambda i: (0, 0)),
        ],
        out_specs=pl.BlockSpec((R, 16), lambda i: (i, 0)),
        out_shape=jax.ShapeDtypeStruct((NP, 16), F32),
    )(p2, xs2, dinv, w1p, b1r, w2p)


def _tc_out(q2, zs16, dinv, b2r):
    def body(q_ref, z_ref, d_ref, b_ref, o_ref):
        s = q_ref[0] + q_ref[1] + z_ref[...]
        o_ref[...] = s * d_ref[...] + b_ref[0, :]

    return pl.pallas_call(
        body,
        grid=(NP // R,),
        in_specs=[
            pl.BlockSpec((2, R, 16), lambda i: (0, i, 0)),
            pl.BlockSpec((R, 16), lambda i: (i, 0)),
            pl.BlockSpec((R, 1), lambda i: (i, 0)),
            pl.BlockSpec((1, 16), lambda i: (0, 0)),
        ],
        out_specs=pl.BlockSpec((R, 16), lambda i: (i, 0)),
        out_shape=jax.ShapeDtypeStruct((NP, 16), F32),
    )(q2, zs16, dinv, b2r)


@jax.jit
def _run(x, edge_index, W1, b1, W2, b2):
    src = edge_index[0].astype(jnp.int32)
    dst = edge_index[1].astype(jnp.int32)
    pad_idx = N + jnp.arange(EPAD - E, dtype=jnp.int32) % (NP - N)
    pad_src = pad_idx
    pad_dst = pad_idx
    srcp = jnp.concatenate([src, pad_src]).reshape(ER, 128)
    dstp = jnp.concatenate([dst, pad_dst]).reshape(ER, 128)
    ed2 = jnp.stack([srcp, dstp], axis=1)
    ed1 = jnp.stack([ed2, jnp.stack([srcp + NP, dstp], axis=1)])

    xpad = jnp.zeros((NP, 32), F32).at[:N, :29].set(x)
    w1p = jnp.zeros((32, 64), F32).at[:29, :].set(W1)
    w2p = jnp.zeros((64, 16), F32).at[:, :2].set(W2)
    b1r = b1.reshape(1, 64)
    b2r = jnp.zeros((1, 16), F32).at[0, :2].set(b2)
    zrow = jnp.zeros((WT, D), F32)
    zdeg = jnp.zeros((WT,), F32)

    degp = _deg_kernel(dstp, zdeg)
    xs2, dinv = _tc_scale(xpad, degp)
    xs_tab = xs2.reshape(2 * NP, 16)
    p2 = _prop1(ed1, xs_tab, zrow)
    zs16 = _tc_mid(p2, xs2, dinv, w1p, b1r, w2p)
    q2 = _prop2(ed2, zs16, zrow)
    outp = _tc_out(q2, zs16, dinv, b2r)
    return outp[:N, :2]


def kernel(x, edge_index, W1, b1, W2, b2):
    return _run(x, edge_index, W1, b1, W2, b2)


# speedup vs baseline: 36.8202x; 1.0053x over previous
import functools

import jax
import jax.numpy as jnp
from jax import lax
from jax.experimental import pallas as pl
from jax.experimental.pallas import tpu as pltpu
from jax.experimental.pallas import tpu_sc as plsc

N = 100000
E = 1600000
NP = 100352
EPAD = 1638400
ER = EPAD // 128
K = 5
WT = NP // 16
D = 16
R = 1024
F32 = jnp.float32

_mesh = plsc.VectorSubcoreMesh(core_axis_name="c", subcore_axis_name="s")
_sc_params = pltpu.CompilerParams(use_tc_tiling_on_sc=False)


@functools.partial(
    pl.kernel,
    out_type=jax.ShapeDtypeStruct((2, NP), F32),
    mesh=_mesh,
    scratch_types=[
        pltpu.VMEM((3, K, 2, 128), jnp.int32),
        pltpu.VMEM((128,), F32),
        pltpu.VMEM((K * 128,), F32),
        pltpu.VMEM_SHARED((NP,), F32),
        pltpu.SemaphoreType.DMA,
        pltpu.SemaphoreType.DMA,
    ],
    compiler_params=_sc_params,
)
def _deg_kernel(ed_hbm, zeros_hbm, out_hbm, edv, ones_v, dsc, acc, isem, ssem):
    cid = lax.axis_index("c")
    sid = lax.axis_index("s")

    def _o16(i, c):
        ones_v[pl.ds(i * 16, 16)] = jnp.ones((16,), F32)
        return c

    lax.fori_loop(0, 8, _o16, 0)

    base = sid * WT
    pltpu.sync_copy(zeros_hbm, acc.at[pl.ds(base, WT)])
    plsc.subcore_barrier()

    rows_per_tile = ER // 32
    nch = rows_per_tile // K
    row_base = cid * (ER // 2) + sid * rows_per_tile

    def start_stage(c, buf):
        pltpu.async_copy(ed_hbm.at[pl.ds(row_base + c * K, K)],
                         edv.at[buf], isem)

    def drain_stage():
        pltpu.make_async_copy(ed_hbm.at[pl.ds(row_base, K)],
                              edv.at[0], isem).wait()

    def start_scatters(buf):
        for j in range(K):
            pltpu.async_copy(ones_v, acc.at[edv.at[buf, j, 1]], ssem, add=True)

    def drain_scatters():
        pltpu.make_async_copy(zeros_hbm.at[pl.ds(0, K * 128)],
                              dsc, ssem).wait()

    start_stage(0, 0)

    def body(ci, carry):
        cb = lax.rem(ci, 3)
        drain_stage()

        @pl.when(ci >= 2)
        def _():
            drain_scatters()

        @pl.when(ci + 1 < nch)
        def _():
            start_stage(ci + 1, lax.rem(ci + 1, 3))

        start_scatters(cb)
        return carry

    lax.fori_loop(0, nch, body, 0)
    drain_scatters()
    drain_scatters()

    plsc.subcore_barrier()
    pltpu.sync_copy(acc.at[pl.ds(base, WT)], out_hbm.at[cid, pl.ds(base, WT)])


def _make_prop(split_edges):
    @functools.partial(
        pl.kernel,
        out_type=jax.ShapeDtypeStruct((2, NP, D), F32),
        mesh=_mesh,
        scratch_types=[
            pltpu.VMEM((3, K, 2, 128), jnp.int32),
            pltpu.VMEM((2, K * 128, D), F32),
            pltpu.VMEM_SHARED((NP, D), F32),
            pltpu.SemaphoreType.DMA,
            pltpu.SemaphoreType.DMA,
            pltpu.SemaphoreType.DMA,
        ],
        compiler_params=_sc_params,
    )
    def prop(ed_hbm, table_hbm, zeros_hbm, out_hbm,
             edv, rows, acc, isem, gsem, ssem):
        cid = lax.axis_index("c")
        sid = lax.axis_index("s")

        base = sid * WT
        pltpu.sync_copy(zeros_hbm, acc.at[pl.ds(base, WT)])
        plsc.subcore_barrier()

        if split_edges:
            rows_per_tile = ER // 32
            row_base = cid * (ER // 2) + sid * rows_per_tile
        else:
            rows_per_tile = ER // 16
            row_base = sid * rows_per_tile

        def ed_slice(r0):
            return ed_hbm.at[pl.ds(r0, K)]

        nch = rows_per_tile // K
        src_off = cid * NP

        def start_stage(c, buf):
            pltpu.async_copy(ed_slice(row_base + c * K), edv.at[buf], isem)

        def drain_stage():
            pltpu.make_async_copy(ed_slice(row_base), edv.at[0], isem).wait()

        def start_gathers(buf3, b2):
            for j in range(K):
                pltpu.async_copy(table_hbm.at[edv.at[buf3, j, 0]],
                                 rows.at[b2, pl.ds(j * 128, 128)], gsem)

        def drain_gathers():
            pltpu.make_async_copy(table_hbm.at[pl.ds(0, K * 128)],
                                  rows.at[0], gsem).wait()

        def start_scatters(buf3, b2):
            for j in range(K):
                pltpu.async_copy(rows.at[b2, pl.ds(j * 128, 128)],
                                 acc.at[edv.at[buf3, j, 1]], ssem, add=True)

        def drain_scatters():
            pltpu.make_async_copy(table_hbm.at[pl.ds(0, K * 128)],
                                  rows.at[0], ssem).wait()

        start_stage(0, 0)

        def body(ci, carry):
            b = lax.rem(ci, 2)
            cb = lax.rem(ci, 3)
            pb = lax.rem(ci + 1, 2)
            pcb = lax.rem(ci + 2, 3)
            drain_stage()
            if not split_edges:
                for j in range(K):
                    for s in range(8):
                        sl = pl.ds(s * 16, 16)
                        edv[cb, j, 0, sl] = edv[cb, j, 0, sl] + src_off

            @pl.when(ci >= 2)
            def _():
                drain_scatters()

            @pl.when(ci + 1 < nch)
            def _():
                start_stage(ci + 1, lax.rem(ci + 1, 3))

            @pl.when(ci >= 1)
            def _():
                drain_gathers()
                start_scatters(pcb, pb)

            start_gathers(cb, b)
            return carry

        lax.fori_loop(0, nch, body, 0)

        drain_scatters()
        drain_gathers()
        start_scatters((nch - 1) % 3, (nch - 1) % 2)
        drain_scatters()

        plsc.subcore_barrier()
        pltpu.sync_copy(acc.at[pl.ds(base, WT)],
                        out_hbm.at[cid, pl.ds(base, WT)])

    return prop


_prop1 = _make_prop(split_edges=False)
_prop2 = _make_prop(split_edges=True)


def _tc_scale(xpad, degp):
    def body(x_ref, d_ref, xs2_ref, dinv_ref):
        deg = d_ref[0, :] + d_ref[1, :] + 1.0
        dinv = lax.rsqrt(deg)
        xs = x_ref[...] * dinv[:, None]
        xs2_ref[0, :, :] = xs[:, :16]
        xs2_ref[1, :, :] = xs[:, 16:]
        dinv_ref[...] = dinv[:, None]

    return pl.pallas_call(
        body,
        grid=(NP // R,),
        in_specs=[
            pl.BlockSpec((R, 32), lambda i: (i, 0)),
            pl.BlockSpec((2, R), lambda i: (0, i)),
        ],
        out_specs=[
            pl.BlockSpec((2, R, 16), lambda i: (0, i, 0)),
            pl.BlockSpec((R, 1), lambda i: (i, 0)),
        ],
        out_shape=[
            jax.ShapeDtypeStruct((2, NP, 16), F32),
            jax.ShapeDtypeStruct((NP, 1), F32),
        ],
    )(xpad, degp)


def _tc_mid(p2, xs2, dinv, w1p, b1r, w2p):
    def body(p_ref, x_ref, d_ref, w1_ref, b1_ref, w2_ref, o_ref):
        t0 = p_ref[0] + x_ref[0]
        t1 = p_ref[1] + x_ref[1]
        tmp = jnp.concatenate([t0, t1], axis=1) * d_ref[...]
        h = jnp.dot(tmp, w1_ref[...], preferred_element_type=F32)
        h = jnp.maximum(h + b1_ref[0, :], 0.0)
        z = jnp.dot(h, w2_ref[...], preferred_element_type=F32)
        o_ref[...] = z * d_ref[...]

    return pl.pallas_call(
        body,
        grid=(NP // R,),
        in_specs=[
            pl.BlockSpec((2, R, 16), lambda i: (0, i, 0)),
            pl.BlockSpec((2, R, 16), lambda i: (0, i, 0)),
            pl.BlockSpec((R, 1), lambda i: (i, 0)),
            pl.BlockSpec((32, 64), lambda i: (0, 0)),
            pl.BlockSpec((1, 64), lambda i: (0, 0)),
            pl.BlockSpec((64, 16), lambda i: (0, 0)),
        ],
        out_specs=pl.BlockSpec((R, 16), lambda i: (i, 0)),
        out_shape=jax.ShapeDtypeStruct((NP, 16), F32),
    )(p2, xs2, dinv, w1p, b1r, w2p)


def _tc_out(q2, zs16, dinv, b2r):
    def body(q_ref, z_ref, d_ref, b_ref, o_ref):
        s = q_ref[0] + q_ref[1] + z_ref[...]
        o_ref[...] = s * d_ref[...] + b_ref[0, :]

    return pl.pallas_call(
        body,
        grid=(NP // R,),
        in_specs=[
            pl.BlockSpec((2, R, 16), lambda i: (0, i, 0)),
            pl.BlockSpec((R, 16), lambda i: (i, 0)),
            pl.BlockSpec((R, 1), lambda i: (i, 0)),
            pl.BlockSpec((1, 16), lambda i: (0, 0)),
        ],
        out_specs=pl.BlockSpec((R, 16), lambda i: (i, 0)),
        out_shape=jax.ShapeDtypeStruct((NP, 16), F32),
    )(q2, zs16, dinv, b2r)


@jax.jit
def _run(x, edge_index, W1, b1, W2, b2):
    src = edge_index[0].astype(jnp.int32)
    dst = edge_index[1].astype(jnp.int32)
    pad_idx = N + jnp.arange(EPAD - E, dtype=jnp.int32) % (NP - N)
    pad_src = pad_idx
    pad_dst = pad_idx
    srcp = jnp.concatenate([src, pad_src]).reshape(ER, 128)
    dstp = jnp.concatenate([dst, pad_dst]).reshape(ER, 128)
    ed2 = jnp.stack([srcp, dstp], axis=1)

    xpad = jnp.zeros((NP, 32), F32).at[:N, :29].set(x)
    w1p = jnp.zeros((32, 64), F32).at[:29, :].set(W1)
    w2p = jnp.zeros((64, 16), F32).at[:, :2].set(W2)
    b1r = b1.reshape(1, 64)
    b2r = jnp.zeros((1, 16), F32).at[0, :2].set(b2)
    zrow = jnp.zeros((WT, D), F32)
    zdeg = jnp.zeros((WT,), F32)

    degp = _deg_kernel(ed2, zdeg)
    xs2, dinv = _tc_scale(xpad, degp)
    xs_tab = xs2.reshape(2 * NP, 16)
    p2 = _prop1(ed2, xs_tab, zrow)
    zs16 = _tc_mid(p2, xs2, dinv, w1p, b1r, w2p)
    q2 = _prop2(ed2, zs16, zrow)
    outp = _tc_out(q2, zs16, dinv, b2r)
    return outp[:N, :2]


def kernel(x, edge_index, W1, b1, W2, b2):
    return _run(x, edge_index, W1, b1, W2, b2)
